# baseline (speedup 1.0000x reference)
import functools

import jax
import jax.numpy as jnp
from jax import lax
from jax.experimental import pallas as pl
from jax.experimental.pallas import tpu as pltpu
from jax.experimental.pallas import tpu_sc as plsc

VOCAB = 1000000
DIM = 64
NC = 2
NS = 16
NW = NC * NS
L = 16
DC = DIM // L
CHUNK = 128


def _sc_body(u_w, v_w, u_pos, v_pos, v_city, v_cntry, out_hbm,
             idx_u, idx_v, idx_c, idx_d, buf_u, buf_v, buf_c, buf_d,
             out_v, sem):
    wid = lax.axis_index("c") * NS + lax.axis_index("s")
    b_per_w = CHUNK * 4
    base = wid * b_per_w

    for j in range(4):
        sl = pl.ds(base + CHUNK * j, CHUNK)
        pltpu.sync_copy(u_pos.at[sl], idx_u.at[j])
        pltpu.sync_copy(v_pos.at[sl], idx_v.at[j])
        pltpu.sync_copy(v_city.at[sl], idx_c.at[j])
        pltpu.sync_copy(v_cntry.at[sl], idx_d.at[j])

    zero = jnp.zeros((L,), jnp.float32)
    accs = (zero,) * (3 * DC)

    for j in range(4):
        h_u = pltpu.async_copy(u_w.at[idx_u.at[j]], buf_u, sem)
        h_v = pltpu.async_copy(v_w.at[idx_v.at[j]], buf_v, sem)
        h_c = pltpu.async_copy(v_w.at[idx_c.at[j]], buf_c, sem)
        h_d = pltpu.async_copy(v_w.at[idx_d.at[j]], buf_d, sem)
        h_u.wait()
        h_v.wait()
        h_c.wait()
        h_d.wait()

        def row_body(r, a):
            a = list(a)
            for c in range(DC):
                sl = pl.ds(c * L, L)
                u = buf_u[r, sl]
                a[c] = a[c] + u * buf_v[r, sl]
                a[DC + c] = a[DC + c] + u * buf_c[r, sl]
                a[2 * DC + c] = a[2 * DC + c] + u * buf_d[r, sl]
            return tuple(a)

        accs = lax.fori_loop(0, CHUNK, row_body, accs)

    for c in range(DC):
        out_v[pl.ds(c * L, L)] = accs[c]
        out_v[pl.ds(DIM + c * L, L)] = -accs[DC + c]
        out_v[pl.ds(2 * DIM + c * L, L)] = -accs[2 * DC + c]
    pltpu.sync_copy(out_v, out_hbm.at[wid])


def _tc_reduce(p_ref, o_ref):
    x = p_ref[...]
    s = jnp.sum(x, axis=0, keepdims=True)
    ls = jnp.minimum(s, 0.0) - jnp.log1p(jnp.exp(-jnp.abs(s)))
    o_ref[0, 0] = -jnp.sum(ls)


def kernel(u_weight, v_weight, u_pos, v_pos, v_neg_city, v_neg_country):
    u_pos = u_pos.astype(jnp.int32)
    v_pos = v_pos.astype(jnp.int32)
    v_neg_city = v_neg_city.astype(jnp.int32)
    v_neg_country = v_neg_country.astype(jnp.int32)

    mesh = plsc.VectorSubcoreMesh(core_axis_name="c", subcore_axis_name="s")
    sc_call = pl.kernel(
        _sc_body,
        out_type=jax.ShapeDtypeStruct((NW, 3 * DIM), jnp.float32),
        mesh=mesh,
        compiler_params=pltpu.CompilerParams(use_tc_tiling_on_sc=False),
        scratch_types=[
            pltpu.VMEM((4, CHUNK), jnp.int32),
            pltpu.VMEM((4, CHUNK), jnp.int32),
            pltpu.VMEM((4, CHUNK), jnp.int32),
            pltpu.VMEM((4, CHUNK), jnp.int32),
            pltpu.VMEM((CHUNK, DIM), jnp.float32),
            pltpu.VMEM((CHUNK, DIM), jnp.float32),
            pltpu.VMEM((CHUNK, DIM), jnp.float32),
            pltpu.VMEM((CHUNK, DIM), jnp.float32),
            pltpu.VMEM((3 * DIM,), jnp.float32),
            pltpu.SemaphoreType.DMA,
        ],
    )
    partials = sc_call(u_weight, v_weight, u_pos, v_pos,
                       v_neg_city, v_neg_country)

    loss = pl.pallas_call(
        _tc_reduce,
        out_shape=jax.ShapeDtypeStruct((1, 1), jnp.float32),
        out_specs=pl.BlockSpec(memory_space=pltpu.SMEM),
    )(partials)
    return loss[0, 0]
